# baseline (speedup 1.0000x reference)
import functools

import jax
import jax.numpy as jnp
from jax.experimental import pallas as pl
from jax.experimental.pallas import tpu as pltpu

B, D, H, W = 16, 64, 512, 512
HBLK = 512

BAR_WIDTH = W // (3 * D + 2)
GAP_WIDTH = (W - D * BAR_WIDTH) // (D + 1)
BEG = ((W - D * BAR_WIDTH) - GAP_WIDTH * (D + 1)) // 2
BAR_START0 = BEG + GAP_WIDTH
BAR_STRIDE = BAR_WIDTH + GAP_WIDTH
ROWS_PER_VALUE = H // D


BBLK = 2


def _tile_kernel(inputs_ref, inputs_t_ref, coords_ref, coords_t_ref,
                 stamp_ref, out_ref):
    f32 = jnp.float32
    dot = functools.partial(jax.lax.dot, precision=jax.lax.Precision.DEFAULT)

    row_c = coords_t_ref[0:1, :]
    col_c = coords_ref[:, 1:2]

    hid = jax.lax.broadcasted_iota(jnp.int32, (HBLK, D), 0)
    did = jax.lax.broadcasted_iota(jnp.int32, (HBLK, D), 1)
    a_t = ((hid // ROWS_PER_VALUE) == did).astype(f32)
    r_1h = (hid == row_c).astype(f32)

    wv = jax.lax.broadcasted_iota(jnp.int32, (D, W), 1)
    d2 = jax.lax.broadcasted_iota(jnp.int32, (D, W), 0)

    c1m = (wv == col_c).astype(f32)
    u3m = ((wv // ROWS_PER_VALUE) == d2).astype(f32)
    bar_off = wv - (BAR_START0 + BAR_STRIDE * d2)
    c4m = ((bar_off >= 0) & (bar_off < BAR_WIDTH)).astype(f32)

    for bb in range(BBLK):
        inp_row = inputs_ref[bb]
        inp_col = inputs_t_ref[bb]

        r_t = r_1h * inp_row

        dist = jnp.abs(inp_col - inp_row)
        mn = jnp.min(dist)
        mx = jnp.max(dist)
        norm = (dist - mn) / (mx - mn)
        norm_rows = dot(a_t, norm)

        bh = jnp.clip(jnp.round(inp_row * H), 0.0, float(H)).astype(jnp.int32)
        rowmask = (hid < bh).astype(f32)

        out_ref[bb, 0] = stamp_ref[...]
        out_ref[bb, 1] = dot(r_t, c1m)
        out_ref[bb, 2] = jnp.broadcast_to(dot(a_t, inp_col), (HBLK, W))
        out_ref[bb, 3] = dot(norm_rows, u3m)
        out_ref[bb, 4] = dot(rowmask, c4m)


@jax.jit
def kernel(inputs, coords, stamp_shape_matrix):
    stamp2d = stamp_shape_matrix.reshape(H, W)
    inputs_r = inputs[:, None, :]
    inputs_c = inputs[:, :, None]
    coords_t = coords.T

    out = pl.pallas_call(
        _tile_kernel,
        grid=(B // BBLK,),
        in_specs=[
            pl.BlockSpec((BBLK, 1, D), lambda b: (b, 0, 0)),
            pl.BlockSpec((BBLK, D, 1), lambda b: (b, 0, 0)),
            pl.BlockSpec((D, 2), lambda b: (0, 0)),
            pl.BlockSpec((2, D), lambda b: (0, 0)),
            pl.BlockSpec((HBLK, W), lambda b: (0, 0)),
        ],
        out_specs=pl.BlockSpec((BBLK, 5, HBLK, W), lambda b: (b, 0, 0, 0)),
        out_shape=jax.ShapeDtypeStruct((B, 5, H, W), jnp.float32),
    )(inputs_r, inputs_c, coords, coords_t, stamp2d)

    return jnp.transpose(out, (0, 2, 3, 1))


# speedup vs baseline: 1.0290x; 1.0290x over previous
import functools

import jax
import jax.numpy as jnp
from jax.experimental import pallas as pl
from jax.experimental.pallas import tpu as pltpu

B, D, H, W = 16, 64, 512, 512
HBLK = 512

BAR_WIDTH = W // (3 * D + 2)
GAP_WIDTH = (W - D * BAR_WIDTH) // (D + 1)
BEG = ((W - D * BAR_WIDTH) - GAP_WIDTH * (D + 1)) // 2
BAR_START0 = BEG + GAP_WIDTH
BAR_STRIDE = BAR_WIDTH + GAP_WIDTH
ROWS_PER_VALUE = H // D


def _tile_kernel(inputs_ref, inputs_t_ref, coords_ref, coords_t_ref,
                 stamp_ref, out_ref):
    h = pl.program_id(0)
    h0 = h * HBLK

    inp_row = inputs_ref[0]
    inp_col = inputs_t_ref[0]
    row_c = coords_t_ref[0:1, :]
    col_c = coords_ref[:, 1:2]

    f32 = jnp.float32
    dot = functools.partial(jax.lax.dot, precision=jax.lax.Precision.DEFAULT)

    hid = jax.lax.broadcasted_iota(jnp.int32, (HBLK, D), 0) + h0
    did = jax.lax.broadcasted_iota(jnp.int32, (HBLK, D), 1)

    r_t = (hid == row_c).astype(f32) * inp_row
    a_t = ((hid // ROWS_PER_VALUE) == did).astype(f32)

    dist = jnp.abs(inp_col - inp_row)
    mn = jnp.min(dist)
    mx = jnp.max(dist)
    norm = (dist - mn) / (mx - mn)
    norm_rows = dot(a_t, norm)

    bh = jnp.clip(jnp.round(inp_row * H), 0.0, float(H)).astype(jnp.int32)
    rowmask = (hid < bh).astype(f32)

    wv = jax.lax.broadcasted_iota(jnp.int32, (D, W), 1)
    d2 = jax.lax.broadcasted_iota(jnp.int32, (D, W), 0)

    c1m = (wv == col_c).astype(f32)
    u3m = ((wv // ROWS_PER_VALUE) == d2).astype(f32)
    bar_off = wv - (BAR_START0 + BAR_STRIDE * d2)
    c4m = ((bar_off >= 0) & (bar_off < BAR_WIDTH)).astype(f32)

    out_ref[0, 0] = stamp_ref[...]
    out_ref[0, 1] = dot(r_t, c1m)
    out_ref[0, 2] = jnp.broadcast_to(dot(a_t, inp_col), (HBLK, W))
    out_ref[0, 3] = dot(norm_rows, u3m)
    out_ref[0, 4] = dot(rowmask, c4m)


@jax.jit
def kernel(inputs, coords, stamp_shape_matrix):
    stamp2d = stamp_shape_matrix.reshape(H, W)
    inputs_r = inputs[:, None, :]
    inputs_c = inputs[:, :, None]
    coords_t = coords.T

    out = pl.pallas_call(
        _tile_kernel,
        grid=(H // HBLK, B),
        in_specs=[
            pl.BlockSpec((1, 1, D), lambda h, b: (b, 0, 0)),
            pl.BlockSpec((1, D, 1), lambda h, b: (b, 0, 0)),
            pl.BlockSpec((D, 2), lambda h, b: (0, 0)),
            pl.BlockSpec((2, D), lambda h, b: (0, 0)),
            pl.BlockSpec((HBLK, W), lambda h, b: (h, 0)),
        ],
        out_specs=pl.BlockSpec((1, 5, HBLK, W), lambda h, b: (b, 0, h, 0)),
        out_shape=jax.ShapeDtypeStruct((B, 5, H, W), jnp.float32),
    )(inputs_r, inputs_c, coords, coords_t, stamp2d)

    return jnp.transpose(out, (0, 2, 3, 1))
